# baseline (speedup 1.0000x reference)
import jax
import jax.numpy as jnp
from jax.experimental import pallas as pl
from jax.experimental.pallas import tpu as pltpu


def _copy_kernel(experts_ref, scales_ref, experts_out_ref, scales_out_ref,
                 experts_vmem, scales_vmem, sem_ei, sem_si, sem_eo, sem_so):
    e_in = pltpu.make_async_copy(experts_ref, experts_vmem, sem_ei)
    s_in = pltpu.make_async_copy(scales_ref, scales_vmem, sem_si)
    e_in.start()
    s_in.start()
    e_in.wait()
    e_out = pltpu.make_async_copy(experts_vmem, experts_out_ref, sem_eo)
    e_out.start()
    s_in.wait()
    s_out = pltpu.make_async_copy(scales_vmem, scales_out_ref, sem_so)
    s_out.start()
    e_out.wait()
    s_out.wait()


def kernel(router_logits, routing_tensor, routing_scales):
    del router_logits
    return pl.pallas_call(
        _copy_kernel,
        in_specs=[
            pl.BlockSpec(memory_space=pl.ANY),
            pl.BlockSpec(memory_space=pl.ANY),
        ],
        out_specs=(
            pl.BlockSpec(memory_space=pl.ANY),
            pl.BlockSpec(memory_space=pl.ANY),
        ),
        out_shape=(
            jax.ShapeDtypeStruct(routing_tensor.shape, routing_tensor.dtype),
            jax.ShapeDtypeStruct(routing_scales.shape, routing_scales.dtype),
        ),
        scratch_shapes=[
            pltpu.VMEM(routing_tensor.shape, routing_tensor.dtype),
            pltpu.VMEM(routing_scales.shape, routing_scales.dtype),
            pltpu.SemaphoreType.DMA,
            pltpu.SemaphoreType.DMA,
            pltpu.SemaphoreType.DMA,
            pltpu.SemaphoreType.DMA,
        ],
    )(routing_tensor, routing_scales)
